# baseline (speedup 1.0000x reference)
import jax
import jax.numpy as jnp
from jax import lax
from jax.experimental import pallas as pl
from jax.experimental.pallas import tpu as pltpu
from jax.experimental.pallas import tpu_sc as plsc

B = 16
IN = 768
C = 8
P = 1024
NW = 32
L = 16
PPW = P // NW
TRW = (B * C) // NW


def _sc_body(x_hbm, w_hbm, idx_hbm, tbl_hbm, conn_hbm,
             x_v, w_v, idx_v, trow_v, conn_v, sem):
    nc = 2
    wid = lax.axis_index("s") * nc + lax.axis_index("c")

    pltpu.sync_copy(x_hbm, x_v)
    pltpu.sync_copy(w_hbm, w_v)
    pltpu.sync_copy(idx_hbm, idx_v)

    p0 = wid * PPW
    for k in range(PPW // L):
        sl = pl.ds(p0 + L * k, L)
        best = w_v[0, sl]
        bc = jnp.zeros((L,), jnp.int32)
        for cand in range(1, C):
            wv = w_v[cand, sl]
            m = wv > best
            best = jnp.where(m, wv, best)
            bc = jnp.where(m, cand, bc)
        conn_v[pl.ds(L * k, L)] = bc
    pltpu.sync_copy(conn_v, conn_hbm.at[pl.ds(p0, PPW)])

    def row_body(j, _):
        r = wid * TRW + j
        b = r // C
        cand = r % C
        bvec = jnp.full((L,), b, jnp.int32)

        def q_body(k, _):
            sl = pl.ds(L * k, L)
            iv = idx_v[cand, sl]
            trow_v[sl] = plsc.load_gather(x_v, [bvec, iv])
            return 0

        lax.fori_loop(0, P // L, q_body, 0)
        pltpu.sync_copy(trow_v, tbl_hbm.at[r])
        return 0

    lax.fori_loop(0, TRW, row_body, 0)


def _tc_body(tbl_ref, conn_ref, out_ref):
    conn = conn_ref[...]
    tbl = tbl_ref[...].reshape(C, P)
    conn2d = jnp.broadcast_to(conn, (P, P))
    out_ref[...] = jnp.take_along_axis(tbl, conn2d, axis=0).reshape(1, P, P)


@jax.jit
def kernel(x, weights, indices):
    w2 = weights.reshape(C, P)
    idx2 = indices.reshape(C, P).astype(jnp.int32)

    mesh = plsc.VectorSubcoreMesh(core_axis_name="c", subcore_axis_name="s")
    tbl, conn = pl.kernel(
        _sc_body,
        out_type=[
            jax.ShapeDtypeStruct((B * C, P), jnp.float32),
            jax.ShapeDtypeStruct((P,), jnp.int32),
        ],
        mesh=mesh,
        scratch_types=[
            pltpu.VMEM((B, IN), jnp.float32),
            pltpu.VMEM((C, P), jnp.float32),
            pltpu.VMEM((C, P), jnp.int32),
            pltpu.VMEM((P,), jnp.float32),
            pltpu.VMEM((PPW,), jnp.int32),
            pltpu.SemaphoreType.DMA,
        ],
        compiler_params=pltpu.CompilerParams(needs_layout_passes=False),
        name="learnable_connections_sc",
    )(x, w2, idx2)

    out3 = pl.pallas_call(
        _tc_body,
        out_shape=jax.ShapeDtypeStruct((B, P, P), jnp.float32),
        grid=(B,),
        in_specs=[
            pl.BlockSpec((1, C, P), lambda b: (b, 0, 0)),
            pl.BlockSpec((P, 1), lambda b: (0, 0)),
        ],
        out_specs=pl.BlockSpec((1, P, P), lambda b: (b, 0, 0)),
        name="learnable_connections_expand_tc",
    )(tbl.reshape(B, C, P), conn.reshape(P, 1))

    return out3.reshape(B, 2, P // 2, 2, P // 2)


# speedup vs baseline: 2.4406x; 2.4406x over previous
import jax
import jax.numpy as jnp
from jax import lax
from jax.experimental import pallas as pl
from jax.experimental.pallas import tpu as pltpu
from jax.experimental.pallas import tpu_sc as plsc

B = 16
IN = 768
C = 8
P = 1024
NW = 32
L = 16
PPW = P // NW
TRW = (B * C) // NW


def _sc_body(x_hbm, w_hbm, idx_hbm, tbl_hbm, conn_hbm,
             x_v, w_v, idx_v, trow_v, conn_v, sem):
    nc = 2
    wid = lax.axis_index("s") * nc + lax.axis_index("c")

    pltpu.sync_copy(x_hbm, x_v)
    pltpu.sync_copy(w_hbm, w_v)
    pltpu.sync_copy(idx_hbm, idx_v)

    p0 = wid * PPW
    for k in range(PPW // L):
        sl = pl.ds(p0 + L * k, L)
        best = w_v[0, sl]
        bc = jnp.zeros((L,), jnp.int32)
        for cand in range(1, C):
            wv = w_v[cand, sl]
            m = wv > best
            best = jnp.where(m, wv, best)
            bc = jnp.where(m, cand, bc)
        conn_v[pl.ds(L * k, L)] = bc
    pltpu.sync_copy(conn_v, conn_hbm.at[pl.ds(p0, PPW)])

    def row_body(j, _):
        r = wid * TRW + j
        b = r // C
        cand = r % C
        bvec = jnp.full((L,), b, jnp.int32)

        def q_body(k, _):
            sl = pl.ds(L * k, L)
            iv = idx_v[cand, sl]
            trow_v[sl] = plsc.load_gather(x_v, [bvec, iv])
            return 0

        lax.fori_loop(0, P // L, q_body, 0)
        pltpu.sync_copy(trow_v, tbl_hbm.at[r])
        return 0

    lax.fori_loop(0, TRW, row_body, 0)


def _tc_body(tbl_ref, conn_ref, out_ref):
    conn = conn_ref[...]
    tbl = tbl_ref[...].reshape(C, P)
    conn2d = jnp.broadcast_to(conn, (P, P))
    expanded = jnp.take_along_axis(tbl, conn2d, axis=0)
    out_ref[...] = expanded.reshape(1, 2, P // 2, 2, P // 2)


@jax.jit
def kernel(x, weights, indices):
    w2 = weights.reshape(C, P)
    idx2 = indices.reshape(C, P).astype(jnp.int32)

    mesh = plsc.VectorSubcoreMesh(core_axis_name="c", subcore_axis_name="s")
    tbl, conn = pl.kernel(
        _sc_body,
        out_type=[
            jax.ShapeDtypeStruct((B * C, P), jnp.float32),
            jax.ShapeDtypeStruct((P,), jnp.int32),
        ],
        mesh=mesh,
        scratch_types=[
            pltpu.VMEM((B, IN), jnp.float32),
            pltpu.VMEM((C, P), jnp.float32),
            pltpu.VMEM((C, P), jnp.int32),
            pltpu.VMEM((P,), jnp.float32),
            pltpu.VMEM((PPW,), jnp.int32),
            pltpu.SemaphoreType.DMA,
        ],
        compiler_params=pltpu.CompilerParams(needs_layout_passes=False),
        name="learnable_connections_sc",
    )(x, w2, idx2)

    out5 = pl.pallas_call(
        _tc_body,
        out_shape=jax.ShapeDtypeStruct((B, 2, P // 2, 2, P // 2), jnp.float32),
        grid=(B,),
        in_specs=[
            pl.BlockSpec((1, C, P), lambda b: (b, 0, 0)),
            pl.BlockSpec((P, 1), lambda b: (0, 0)),
        ],
        out_specs=pl.BlockSpec((1, 2, P // 2, 2, P // 2), lambda b: (b, 0, 0, 0, 0)),
        name="learnable_connections_expand_tc",
    )(tbl.reshape(B, C, P), conn.reshape(P, 1))

    return out5


# speedup vs baseline: 2.6386x; 1.0811x over previous
import jax
import jax.numpy as jnp
from jax import lax
from jax.experimental import pallas as pl
from jax.experimental.pallas import tpu as pltpu
from jax.experimental.pallas import tpu_sc as plsc

B = 16
IN = 768
C = 8
P = 1024
NW = 32
L = 16
PPW = P // NW
TRW = (B * C) // NW


def _sc_body(x_hbm, w_hbm, idx_hbm, tbl_hbm, conn_hbm,
             xb_v, ws_v, idx_v, trow_v, conn_v, sem_x, sem_w, sem_i, sem):
    nc = 2
    wid = lax.axis_index("s") * nc + lax.axis_index("c")
    b = wid // 2
    cand0 = (wid % 2) * TRW
    p0 = wid * PPW

    cx = pltpu.make_async_copy(x_hbm.at[b], xb_v, sem_x)
    cw = pltpu.make_async_copy(w_hbm, ws_v, sem_w)
    ci = pltpu.make_async_copy(idx_hbm.at[pl.ds(cand0, TRW)], idx_v, sem_i)
    cx.start(); cw.start(); ci.start()
    cw.wait()

    for k in range(PPW // L):
        sl = pl.ds(p0 + L * k, L)
        best = ws_v[0, sl]
        bc = jnp.zeros((L,), jnp.int32)
        for cand in range(1, C):
            wv = ws_v[cand, sl]
            m = wv > best
            best = jnp.where(m, wv, best)
            bc = jnp.where(m, cand, bc)
        conn_v[pl.ds(L * k, L)] = bc
    cc = pltpu.make_async_copy(conn_v, conn_hbm.at[pl.ds(p0, PPW)], sem)
    cc.start()
    cx.wait()
    ci.wait()

    row_copies = []
    for j in range(TRW):
        def q_body(k, _, j=j):
            for u in range(4):
                sl = pl.ds(4 * L * k + L * u, L)
                iv = idx_v[j, sl]
                trow_v[j, sl] = plsc.load_gather(xb_v, [iv])
            return 0

        lax.fori_loop(0, P // (4 * L), q_body, 0)
        cr = pltpu.make_async_copy(trow_v.at[j], tbl_hbm.at[wid * TRW + j], sem)
        cr.start()
        row_copies.append(cr)
    cc.wait()
    for cr in row_copies:
        cr.wait()


def _tc_body(tbl_ref, conn_ref, out_ref):
    conn = conn_ref[...]
    tbl = tbl_ref[...].reshape(C, P)
    conn2d = jnp.broadcast_to(conn, (P, P))
    expanded = jnp.take_along_axis(tbl, conn2d, axis=0)
    out_ref[...] = expanded.reshape(1, 2, P // 2, 2, P // 2)


@jax.jit
def kernel(x, weights, indices):
    w2 = weights.reshape(C, P)
    idx2 = indices.reshape(C, P).astype(jnp.int32)

    mesh = plsc.VectorSubcoreMesh(core_axis_name="c", subcore_axis_name="s")
    tbl, conn = pl.kernel(
        _sc_body,
        out_type=[
            jax.ShapeDtypeStruct((B * C, P), jnp.float32),
            jax.ShapeDtypeStruct((P,), jnp.int32),
        ],
        mesh=mesh,
        scratch_types=[
            pltpu.VMEM((IN,), jnp.float32),
            pltpu.VMEM((C, P), jnp.float32),
            pltpu.VMEM((TRW, P), jnp.int32),
            pltpu.VMEM((TRW, P), jnp.float32),
            pltpu.VMEM((PPW,), jnp.int32),
            pltpu.SemaphoreType.DMA,
            pltpu.SemaphoreType.DMA,
            pltpu.SemaphoreType.DMA,
            pltpu.SemaphoreType.DMA,
        ],
        compiler_params=pltpu.CompilerParams(needs_layout_passes=False),
        name="learnable_connections_sc",
    )(x, w2, idx2)

    out5 = pl.pallas_call(
        _tc_body,
        out_shape=jax.ShapeDtypeStruct((B, 2, P // 2, 2, P // 2), jnp.float32),
        grid=(B,),
        in_specs=[
            pl.BlockSpec((1, C, P), lambda b: (b, 0, 0)),
            pl.BlockSpec((P, 1), lambda b: (0, 0)),
        ],
        out_specs=pl.BlockSpec((1, 2, P // 2, 2, P // 2), lambda b: (b, 0, 0, 0, 0)),
        name="learnable_connections_expand_tc",
    )(tbl.reshape(B, C, P), conn.reshape(P, 1))

    return out5
